# Initial kernel scaffold; baseline (speedup 1.0000x reference)
import jax
import jax.numpy as jnp
from jax.experimental import pallas as pl


def kernel(means3D, means2D, opacities, colors_precomp, scales, rotations):
    raise NotImplementedError("write your pallas kernel here")


# speedup vs baseline: 2.9777x; 2.9777x over previous
import jax
import jax.numpy as jnp
from jax.experimental import pallas as pl
from jax.experimental.pallas import tpu as pltpu

N = 2048
H = 64
W = 64
TANFOVX = 0.5
TANFOVY = 0.5
NPIX = H * W

G = 256
NC = N // G
PT = 1024
NPT = NPIX // PT

_INV_LN2 = 1.4426950408889634
_ALPHA_MIN = 1.0 / 255.0


def _dot_t(a, b):
    return jax.lax.dot_general(a, b, (((1,), (1,)), ((), ())),
                               preferred_element_type=jnp.float32,
                               precision=jax.lax.Precision.HIGHEST)


def _dot_lt(a, b):
    return jax.lax.dot_general(a, b, (((0,), (0,)), ((), ())),
                               preferred_element_type=jnp.float32,
                               precision=jax.lax.Precision.HIGHEST)


def _raster_kernel(m3T, zcol, opT, colT, scT, rotT,
                   color_out, invd_out, radii_out,
                   attrs, T_scr, acc_scr):
    gc = pl.program_id(0)
    pt = pl.program_id(1)

    @pl.when((gc == 0) & (pt == 0))
    def _preprocess():
        fx = W / (2.0 * TANFOVX)
        fy = H / (2.0 * TANFOVY)
        mx = m3T[0:1, :]
        my = m3T[1:2, :]
        mz = m3T[2:3, :]
        tz = jnp.maximum(mz, 1e-4)
        itz = 1.0 / tz
        txtz = jnp.clip(mx * itz, -1.3 * TANFOVX, 1.3 * TANFOVX) * tz
        tytz = jnp.clip(my * itz, -1.3 * TANFOVY, 1.3 * TANFOVY) * tz

        qw = rotT[0:1, :]
        qx = rotT[1:2, :]
        qy = rotT[2:3, :]
        qz = rotT[3:4, :]
        qn = jnp.sqrt(qw * qw + qx * qx + qy * qy + qz * qz)
        w_ = qw / qn
        x_ = qx / qn
        y_ = qy / qn
        z_ = qz / qn
        r00 = 1.0 - 2.0 * (y_ * y_ + z_ * z_)
        r01 = 2.0 * (x_ * y_ - w_ * z_)
        r02 = 2.0 * (x_ * z_ + w_ * y_)
        r10 = 2.0 * (x_ * y_ + w_ * z_)
        r11 = 1.0 - 2.0 * (x_ * x_ + z_ * z_)
        r12 = 2.0 * (y_ * z_ - w_ * x_)
        r20 = 2.0 * (x_ * z_ - w_ * y_)
        r21 = 2.0 * (y_ * z_ + w_ * x_)
        r22 = 1.0 - 2.0 * (x_ * x_ + y_ * y_)

        s0 = scT[0:1, :] ** 2
        s1 = scT[1:2, :] ** 2
        s2 = scT[2:3, :] ** 2
        S00 = s0 * r00 * r00 + s1 * r01 * r01 + s2 * r02 * r02
        S01 = s0 * r00 * r10 + s1 * r01 * r11 + s2 * r02 * r12
        S02 = s0 * r00 * r20 + s1 * r01 * r21 + s2 * r02 * r22
        S11 = s0 * r10 * r10 + s1 * r11 * r11 + s2 * r12 * r12
        S12 = s0 * r10 * r20 + s1 * r11 * r21 + s2 * r12 * r22
        S22 = s0 * r20 * r20 + s1 * r21 * r21 + s2 * r22 * r22

        a0 = fx * itz
        a2 = -fx * txtz * itz * itz
        b1 = fy * itz
        b2 = -fy * tytz * itz * itz
        cov00 = a0 * a0 * S00 + 2.0 * a0 * a2 * S02 + a2 * a2 * S22
        cov01 = a0 * b1 * S01 + a2 * b1 * S12 + a0 * b2 * S02 + a2 * b2 * S22
        cov11 = b1 * b1 * S11 + 2.0 * b1 * b2 * S12 + b2 * b2 * S22
        a_ = cov00 + 0.3
        b_ = cov01
        c_ = cov11 + 0.3
        det = jnp.maximum(a_ * c_ - b_ * b_, 1e-12)
        idet = 1.0 / det
        A = (-0.5 * _INV_LN2) * (c_ * idet)
        Bq = _INV_LN2 * (b_ * idet)
        C = (-0.5 * _INV_LN2) * (a_ * idet)

        px = (mx * itz / TANFOVX + 1.0) * (W * 0.5) - 0.5
        py = (my * itz / TANFOVY + 1.0) * (H * 0.5) - 0.5

        def bb(v):
            return v.astype(jnp.bfloat16).astype(jnp.float32)

        sr0 = scT[0:1, :]
        sr1 = scT[1:2, :]
        sr2 = scT[2:3, :]
        M00 = bb(r00 * sr0); M01 = bb(r01 * sr1); M02 = bb(r02 * sr2)
        M10 = bb(r10 * sr0); M11 = bb(r11 * sr1); M12 = bb(r12 * sr2)
        M20 = bb(r20 * sr0); M21 = bb(r21 * sr1); M22 = bb(r22 * sr2)
        Sg00 = M00 * M00 + M01 * M01 + M02 * M02
        Sg01 = M00 * M10 + M01 * M11 + M02 * M12
        Sg02 = M00 * M20 + M01 * M21 + M02 * M22
        Sg11 = M10 * M10 + M11 * M11 + M12 * M12
        Sg12 = M10 * M20 + M11 * M21 + M12 * M22
        Sg22 = M20 * M20 + M21 * M21 + M22 * M22
        z2 = tz * tz
        j00 = bb(fx / tz)
        j02 = bb(-fx * txtz / z2)
        j11 = bb(fy / tz)
        j12 = bb(-fy * tytz / z2)
        Sb00 = bb(Sg00); Sb01 = bb(Sg01); Sb02 = bb(Sg02)
        Sb11 = bb(Sg11); Sb12 = bb(Sg12); Sb22 = bb(Sg22)
        JS00 = bb(j00 * Sb00 + j02 * Sb02)
        JS01 = bb(j00 * Sb01 + j02 * Sb12)
        JS02 = bb(j00 * Sb02 + j02 * Sb22)
        JS10 = bb(j11 * Sb01 + j12 * Sb02)
        JS11 = bb(j11 * Sb11 + j12 * Sb12)
        JS12 = bb(j11 * Sb12 + j12 * Sb22)
        cov00r = JS00 * j00 + JS02 * j02
        cov01r = JS01 * j11 + JS02 * j12
        cov11r = JS11 * j11 + JS12 * j12
        a_r = cov00r + 0.3
        b_r = cov01r
        c_r = cov11r + 0.3
        det_r = jnp.maximum(a_r * c_r - b_r * b_r, 1e-12)
        mid = 0.5 * (a_r + c_r)
        lam1 = mid + jnp.sqrt(jnp.maximum(0.1, mid * mid - det_r))
        radii_out[0:1, :] = jnp.ceil(3.0 * jnp.sqrt(lam1)).astype(jnp.int32)

        opv = jnp.where(mz > 0.2, opT[0:1, :], 0.0)
        zero_row = jnp.zeros((1, N), jnp.float32)
        attr16 = jnp.concatenate(
            [px, py, A, Bq, C, opv, zero_row, zero_row,
             colT[0:1, :], colT[1:2, :], colT[2:3, :], itz,
             zero_row, zero_row, zero_row, zero_row], axis=0)

        j_iota = jax.lax.broadcasted_iota(jnp.int32, (1, N), 1)
        RB = 256

        def rank_body(i, r):
            zb = zcol[pl.ds(i * RB, RB), :]
            ib = jax.lax.broadcasted_iota(jnp.int32, (RB, 1), 0) + i * RB
            before = (zb < mz) | ((zb == mz) & (ib < j_iota))
            return r + jnp.sum(before.astype(jnp.float32), axis=0,
                               keepdims=True)

        rnk = jax.lax.fori_loop(0, N // RB, rank_body,
                                jnp.zeros((1, N), jnp.float32))

        def sort_body(b, _):
            rio = (jax.lax.broadcasted_iota(jnp.int32, (RB, 1), 0)
                   + b * RB).astype(jnp.float32)
            P = (rnk == rio).astype(jnp.float32)
            attrs[pl.ds(b * RB, RB), :] = _dot_t(P, attr16)
            return 0

        jax.lax.fori_loop(0, N // RB, sort_body, 0)

    ch = attrs[pl.ds(gc * G, G), :]
    px = ch[:, 0:1]
    py = ch[:, 1:2]
    A = ch[:, 2:3]
    Bq = ch[:, 3:4]
    C = ch[:, 4:5]
    op = ch[:, 5:6]
    colz = ch[:, 8:12]

    lio = jax.lax.broadcasted_iota(jnp.int32, (1, PT), 1) + pt * PT
    xs = (lio & (W - 1)).astype(jnp.float32)
    ys = (lio >> 6).astype(jnp.float32)
    dx = px - xs
    dy = py - ys
    p2 = (A * dx + Bq * dy) * dx + C * (dy * dy)
    Gv = jnp.exp2(jnp.minimum(p2, 0.0))
    al = jnp.minimum(0.99, op * Gv)
    al = jnp.where((p2 > 0.0) | (al < _ALPHA_MIN), 0.0, al)

    cp = 1.0 - al
    d = 1
    while d < G:
        cp = cp * jnp.concatenate(
            [jnp.ones((d, PT), jnp.float32), cp[:G - d]], axis=0)
        d *= 2
    exc = jnp.concatenate(
        [jnp.ones((1, PT), jnp.float32), cp[:G - 1]], axis=0)

    Tc = jnp.where(gc == 0, jnp.ones((1, PT), jnp.float32), T_scr[pt])
    w = al * (exc * Tc)
    contrib = _dot_lt(colz, w)
    accs = jnp.where(gc == 0, contrib, acc_scr[pt] + contrib)
    acc_scr[pt] = accs
    T_scr[pt] = Tc * cp[G - 1:G]

    @pl.when(gc == NC - 1)
    def _emit():
        color_out[:, pl.ds(pt * PT, PT)] = accs[0:3, :]
        invd_out[:, pl.ds(pt * PT, PT)] = accs[3:4, :]


def _run(m3T, zcol, opT, colT, scT, rotT, interpret=False):
    fullspec = lambda arr: pl.BlockSpec(arr.shape, lambda gc, pt: (0, 0))
    return pl.pallas_call(
        _raster_kernel,
        grid=(NC, NPT),
        in_specs=[fullspec(m3T), fullspec(zcol), fullspec(opT),
                  fullspec(colT), fullspec(scT), fullspec(rotT)],
        out_specs=[pl.BlockSpec((3, NPIX), lambda gc, pt: (0, 0)),
                   pl.BlockSpec((1, NPIX), lambda gc, pt: (0, 0)),
                   pl.BlockSpec((1, N), lambda gc, pt: (0, 0))],
        out_shape=[jax.ShapeDtypeStruct((3, NPIX), jnp.float32),
                   jax.ShapeDtypeStruct((1, NPIX), jnp.float32),
                   jax.ShapeDtypeStruct((1, N), jnp.int32)],
        scratch_shapes=[pltpu.VMEM((N, 16), jnp.float32),
                        pltpu.VMEM((NPT, 1, PT), jnp.float32),
                        pltpu.VMEM((NPT, 4, PT), jnp.float32)],
        interpret=interpret,
    )(m3T, zcol, opT, colT, scT, rotT)


def kernel(means3D, means2D, opacities, colors_precomp, scales, rotations):
    del means2D
    m3T = means3D.T
    zcol = means3D[:, 2:3]
    opT = opacities.T
    colT = colors_precomp.T
    scT = scales.T
    rotT = rotations.T
    color_f, invd_f, radii2 = _run(m3T, zcol, opT, colT, scT, rotT)
    color = color_f.reshape(3, H, W)
    invd = invd_f.reshape(1, H, W)
    radii = radii2.reshape(N)
    return color, radii, invd
